# baseline (speedup 1.0000x reference)
import functools

import jax
import jax.numpy as jnp
from jax.experimental import pallas as pl
from jax.experimental.pallas import tpu as pltpu


def _banded_tile_weights(w_oihw, positions):
    C_out, C_in, KH, KW = w_oihw.shape
    mats = []
    for kh in range(KH):
        m = jnp.zeros((2 * positions * C_in, positions * C_out), jnp.float32)
        for kw in range(KW):
            sel = jnp.eye(2 * positions, positions, k=-kw, dtype=jnp.float32)
            m = m + jnp.kron(sel, w_oihw[:, :, kh, kw].T)
        mats.append(m)
    return jnp.stack(mats)


def _fused_kernel(x_ref, w1_ref, w2_ref, b_ref, o_ref, h_ref, *, B, H, T, C):
    R = H + 2
    M = B * R - 2
    LT = 128
    wl = T * LT

    zrow = jnp.zeros((1, h_ref.shape[1]), jnp.bfloat16)
    for b in range(B):
        h_ref[R * b:R * b + 1, :] = zrow
        h_ref[R * b + R - 1:R * b + R, :] = zrow
    h_ref[:, 0:C] = jnp.zeros((B * R, C), jnp.bfloat16)
    h_ref[:, C + wl:] = jnp.zeros((B * R, h_ref.shape[1] - C - wl),
                                  jnp.bfloat16)

    b1v = b_ref[0:1, :]
    b2v = b_ref[1:2, :]

    for t in range(T):
        acc = jnp.zeros((M, LT), jnp.float32)
        for di in range(3):
            acc = acc + jnp.dot(x_ref[0, di:di + M, LT * t:LT * t + 2 * LT],
                                w1_ref[di],
                                preferred_element_type=jnp.float32)
        hv = jnp.maximum(acc + b1v, 0.0).astype(jnp.bfloat16)
        for b in range(B):
            h_ref[R * b + 1:R * b + 1 + H, C + LT * t:C + LT * t + LT] = (
                hv[R * b:R * b + H, :])

    for t in range(T):
        acc = jnp.zeros((M, LT), jnp.float32)
        for di in range(3):
            acc = acc + jnp.dot(h_ref[di:di + M, LT * t:LT * t + 2 * LT],
                                w2_ref[di],
                                preferred_element_type=jnp.float32)
        ov = jnp.maximum(acc + b2v, 0.0)
        for b in range(B):
            o_ref[0, b, :, LT * t:LT * t + LT] = ov[R * b:R * b + H, :]


def kernel(x_nchw, w1, b1, w2, b2):
    N, C_in, H, W = x_nchw.shape
    C = C_in
    P = 128 // C
    T = (W * C) // 128
    B = 8
    R = H + 2
    padded = (T + 1) * 128
    pad_right = padded - C - W * C

    x = jax.lax.reshape(x_nchw, (N, H, W * C), dimensions=(0, 2, 3, 1))
    xp = jnp.pad(x, ((0, 0), (1, 1), (C, pad_right))).astype(jnp.bfloat16)
    xp = xp.reshape(N // B, B * R, padded)

    wt1 = _banded_tile_weights(w1, P).astype(jnp.bfloat16)
    wt2 = _banded_tile_weights(w2, P).astype(jnp.bfloat16)
    bb = jnp.stack([jnp.tile(b1.astype(jnp.float32), P),
                    jnp.tile(b2.astype(jnp.float32), P)])

    _body = functools.partial(_fused_kernel, B=B, H=H, T=T, C=C)

    out = pl.pallas_call(
        _body,
        out_shape=jax.ShapeDtypeStruct((N // B, B, H, W * C), jnp.float32),
        grid_spec=pltpu.PrefetchScalarGridSpec(
            num_scalar_prefetch=0,
            grid=(N // B,),
            in_specs=[
                pl.BlockSpec((1, B * R, padded), lambda g: (g, 0, 0)),
                pl.BlockSpec((3, 256, 128), lambda g: (0, 0, 0)),
                pl.BlockSpec((3, 256, 128), lambda g: (0, 0, 0)),
                pl.BlockSpec((2, 128), lambda g: (0, 0)),
            ],
            out_specs=pl.BlockSpec((1, B, H, W * C), lambda g: (g, 0, 0, 0)),
            scratch_shapes=[pltpu.VMEM((B * R, padded), jnp.bfloat16)],
        ),
        compiler_params=pltpu.CompilerParams(
            dimension_semantics=("parallel",),
            vmem_limit_bytes=64 * 1024 * 1024,
        ),
    )(xp, wt1, wt2, bb)

    return jnp.transpose(out.reshape(N, H, W, C), (0, 3, 1, 2))


# speedup vs baseline: 1.9319x; 1.9319x over previous
import functools

import jax
import jax.numpy as jnp
from jax.experimental import pallas as pl
from jax.experimental.pallas import tpu as pltpu


def _tap_stack(v, H, W, taps_ref):
    C, L = v.shape
    l = jax.lax.broadcasted_iota(jnp.int32, (1, L), 1)
    wpos = jax.lax.rem(l, W)
    for kh in range(3):
        for kw in range(3):
            delta = W * (kh - 1) + (kw - 1)
            t = jnp.roll(v, -delta, axis=1) if delta else v
            mask = None
            if kw == 0:
                mask = wpos != 0
            elif kw == 2:
                mask = wpos != W - 1
            if kh == 0:
                mh = l >= W
                mask = mh if mask is None else (mask & mh)
            elif kh == 2:
                mh = l < L - W
                mask = mh if mask is None else (mask & mh)
            if mask is not None:
                t = jnp.where(mask, t, jnp.bfloat16(0))
            taps_ref[C * (3 * kh + kw):C * (3 * kh + kw + 1), :] = t


def _planar_kernel(x_ref, w1_ref, w2_ref, b_ref, o_ref, t_ref, *, B, H, W):
    b1c = b_ref[:, 0:1]
    b2c = b_ref[:, 1:2]
    w1v = w1_ref[...]
    w2v = w2_ref[...]
    for b in range(B):
        xb = x_ref[b].astype(jnp.bfloat16)
        _tap_stack(xb, H, W, t_ref)
        a1 = jnp.dot(w1v, t_ref[...], preferred_element_type=jnp.float32)
        h1 = jnp.maximum(a1 + b1c, 0.0).astype(jnp.bfloat16)
        _tap_stack(h1, H, W, t_ref)
        a2 = jnp.dot(w2v, t_ref[...], preferred_element_type=jnp.float32)
        o_ref[b] = jnp.maximum(a2 + b2c, 0.0)


def kernel(x_nchw, w1, b1, w2, b2):
    N, C, H, W = x_nchw.shape
    B = 8
    L = H * W

    xv = x_nchw.reshape(N, C, L)
    w1p = jnp.transpose(w1, (0, 2, 3, 1)).reshape(C, 9 * C)
    w2p = jnp.transpose(w2, (0, 2, 3, 1)).reshape(C, 9 * C)
    bb = jnp.stack([b1.astype(jnp.float32), b2.astype(jnp.float32)], axis=1)

    _body = functools.partial(_planar_kernel, B=B, H=H, W=W)

    out = pl.pallas_call(
        _body,
        out_shape=jax.ShapeDtypeStruct((N, C, L), jnp.float32),
        grid_spec=pltpu.PrefetchScalarGridSpec(
            num_scalar_prefetch=0,
            grid=(N // B,),
            in_specs=[
                pl.BlockSpec((B, C, L), lambda g: (g, 0, 0)),
                pl.BlockSpec((C, 9 * C), lambda g: (0, 0)),
                pl.BlockSpec((C, 9 * C), lambda g: (0, 0)),
                pl.BlockSpec((C, 2), lambda g: (0, 0)),
            ],
            out_specs=pl.BlockSpec((B, C, L), lambda g: (g, 0, 0)),
            scratch_shapes=[pltpu.VMEM((9 * C, L), jnp.bfloat16)],
        ),
        compiler_params=pltpu.CompilerParams(
            dimension_semantics=("parallel",),
            vmem_limit_bytes=64 * 1024 * 1024,
        ),
    )(xv, w1p.astype(jnp.bfloat16), w2p.astype(jnp.bfloat16), bb)

    return out.reshape(N, C, H, W)
